# baseline (speedup 1.0000x reference)
import functools

import jax
import jax.numpy as jnp
from jax import lax
from jax.experimental import pallas as pl
from jax.experimental.pallas import tpu as pltpu
from jax.experimental.pallas import tpu_sc as plsc

_NC = 2
_NS = 16
_NW = _NC * _NS
_L = 16


def _sc_tail_body(x_hbm, out_hbm, buf0, buf1, osum, sem0, sem1, *,
                  head_rows, rows_pw, chunk_rows, d):
    wid = lax.axis_index("s") * _NC + lax.axis_index("c")
    base_row = head_rows + wid * rows_pw
    nchunks = rows_pw // chunk_rows
    bufs = (buf0, buf1)
    sems = (sem0, sem1)

    for b in range(2):
        pltpu.make_async_copy(
            x_hbm.at[pl.ds(base_row + b * chunk_rows, chunk_rows), :],
            bufs[b], sems[b]).start()

    lane_ids = lax.iota(jnp.int32, _L)

    def outer(it, _):
        for b in range(2):
            c = 2 * it + b
            row0 = c * chunk_rows
            pltpu.make_async_copy(
                x_hbm.at[pl.ds(base_row + row0, chunk_rows), :],
                bufs[b], sems[b]).wait()

            def gbody(g, _, _b=b, _row0=row0):
                row_idx = g * _L + lane_ids

                def jbody(jq, acc, _ridx=row_idx):
                    for u in range(4):
                        col = jnp.full((_L,), jq * 4 + u, jnp.int32)
                        v = plsc.load_gather(bufs[_b], [_ridx, col])
                        acc = acc + v * v
                    return acc

                acc = lax.fori_loop(0, d // 4, jbody,
                                    jnp.zeros((_L,), jnp.float32))
                osum[pl.ds(_row0 + g * _L, _L)] = acc
                return 0

            lax.fori_loop(0, chunk_rows // _L, gbody, 0)

            @pl.when(c + 2 < nchunks)
            def _prefetch(_b=b, _c=c):
                pltpu.make_async_copy(
                    x_hbm.at[pl.ds(base_row + (_c + 2) * chunk_rows, chunk_rows), :],
                    bufs[_b], sems[_b]).start()
        return 0

    lax.fori_loop(0, nchunks // 2, outer, 0)
    pltpu.sync_copy(osum, out_hbm.at[pl.ds(wid * rows_pw, rows_pw)])


def _tc_head_body(x_ref, xm_ref, sq_ref, df_ref):
    x = x_ref[...]
    dx = x - xm_ref[...]
    r8 = x.shape[0] // 128
    sq_ref[...] = jnp.sum(x * x, axis=1).reshape(r8, 128)
    df_ref[...] = jnp.sum(dx * dx, axis=1).reshape(r8, 128)


def _tc_final_body(sqh_ref, sqt_ref, df_ref, out_ref, *, n, k, d):
    sq_all = jnp.concatenate([sqh_ref[...], sqt_ref[...]], axis=0)
    norms = jnp.sqrt(sq_all)
    total_norm = jnp.sum(norms)
    fro2 = jnp.sum(sq_all)
    bits = lax.bitcast_convert_type(norms, jnp.int32)

    def bisect(_, lohi):
        lo, hi = lohi
        mid = lo + (hi - lo) // 2
        cnt = jnp.sum((bits >= mid).astype(jnp.int32))
        ge = cnt >= k
        return (jnp.where(ge, mid, lo), jnp.where(ge, hi, mid))

    lo, _ = lax.fori_loop(0, 31, bisect, (jnp.int32(0), jnp.int32(0x7F800000)))
    tval = jnp.max(jnp.where(bits <= lo, norms, 0.0))
    gt = bits > lo
    cnt_gt = jnp.sum(gt.astype(jnp.int32))
    sum_gt = jnp.sum(jnp.where(gt, norms, 0.0))
    top_sum = sum_gt + (k - cnt_gt).astype(jnp.float32) * tval

    gamma = top_sum / total_norm
    thr = (1.0 - gamma) * (1.0 - gamma) * fro2
    inv_d = 1.0 / d
    pen1 = jnp.sum(jnp.maximum(df_ref[...] * inv_d - thr, 0.0))
    pen2 = jnp.sum(jnp.maximum(sqt_ref[...] * inv_d - thr, 0.0))
    out_ref[0, 0] = (pen1 + pen2) / n


def kernel(X, X_merged):
    B, N, D = X.shape
    K = X_merged.shape[1]
    top_k = min(K, N // 2)
    TAIL = N - K
    ROWS_PW = TAIL // _NW
    CHUNK = 32

    X2 = X.reshape(N, D)
    Xm2 = X_merged.reshape(K, D)

    mesh = plsc.VectorSubcoreMesh(core_axis_name="c", subcore_axis_name="s")
    sc_tail = functools.partial(
        pl.kernel,
        mesh=mesh,
        compiler_params=pltpu.CompilerParams(
            needs_layout_passes=False),
        out_type=jax.ShapeDtypeStruct((TAIL,), jnp.float32),
        scratch_types=[
            pltpu.VMEM((CHUNK, D), jnp.float32),
            pltpu.VMEM((CHUNK, D), jnp.float32),
            pltpu.VMEM((ROWS_PW,), jnp.float32),
            pltpu.SemaphoreType.DMA,
            pltpu.SemaphoreType.DMA,
        ],
    )(functools.partial(_sc_tail_body, head_rows=K, rows_pw=ROWS_PW,
                        chunk_rows=CHUNK, d=D))
    sq_tail = sc_tail(X2)

    HB = 1024
    sq_head, df = pl.pallas_call(
        _tc_head_body,
        grid=(K // HB,),
        in_specs=[
            pl.BlockSpec((HB, D), lambda i: (i, 0)),
            pl.BlockSpec((HB, D), lambda i: (i, 0)),
        ],
        out_specs=[
            pl.BlockSpec((HB // 128, 128), lambda i: (i, 0)),
            pl.BlockSpec((HB // 128, 128), lambda i: (i, 0)),
        ],
        out_shape=[
            jax.ShapeDtypeStruct((K // 128, 128), jnp.float32),
            jax.ShapeDtypeStruct((K // 128, 128), jnp.float32),
        ],
    )(X2, Xm2)

    final_body = functools.partial(_tc_final_body, n=N, k=top_k, d=float(D))
    out = pl.pallas_call(
        final_body,
        out_specs=pl.BlockSpec(memory_space=pltpu.SMEM),
        out_shape=jax.ShapeDtypeStruct((1, 1), jnp.float32),
    )(sq_head, sq_tail.reshape(TAIL // 128, 128), df)
    return out.reshape(())


# speedup vs baseline: 2.3104x; 2.3104x over previous
import functools

import jax
import jax.numpy as jnp
from jax import lax
from jax.experimental import pallas as pl
from jax.experimental.pallas import tpu as pltpu
from jax.experimental.pallas import tpu_sc as plsc

_NC = 2
_NS = 16
_NW = _NC * _NS
_L = 16


def _sc_tail_body(x_hbm, out_hbm, buf0, buf1, osum, sem0, sem1, *,
                  head_rows, rows_pw, chunk_rows, d):
    wid = lax.axis_index("s") * _NC + lax.axis_index("c")
    base_row = head_rows + wid * rows_pw
    nchunks = rows_pw // chunk_rows
    bufs = (buf0, buf1)
    sems = (sem0, sem1)

    for b in range(2):
        pltpu.make_async_copy(
            x_hbm.at[pl.ds(base_row + b * chunk_rows, chunk_rows), :],
            bufs[b], sems[b]).start()

    lane_ids = lax.iota(jnp.int32, _L)

    def outer(it, _):
        for b in range(2):
            c = 2 * it + b
            row0 = c * chunk_rows
            pltpu.make_async_copy(
                x_hbm.at[pl.ds(base_row + row0, chunk_rows), :],
                bufs[b], sems[b]).wait()

            def gbody(g, _, _b=b, _row0=row0):
                rowsums = jnp.zeros((_L,), jnp.float32)
                for r in range(_L):
                    rr = g * _L + r
                    acc = jnp.zeros((_L,), jnp.float32)

                    def jbody(jq, acc, _rr=rr):
                        off = jq * (_L * 32)
                        for u in range(32):
                            v = bufs[_b][_rr, pl.ds(off + u * _L, _L)]
                            acc = acc + v * v
                        return acc

                    acc = lax.fori_loop(0, d // (_L * 32), jbody, acc)
                    rowsums = jnp.where(lane_ids == r, jnp.sum(acc), rowsums)
                osum[pl.ds(_row0 + g * _L, _L)] = rowsums
                return 0

            lax.fori_loop(0, chunk_rows // _L, gbody, 0)

            @pl.when(c + 2 < nchunks)
            def _prefetch(_b=b, _c=c):
                pltpu.make_async_copy(
                    x_hbm.at[pl.ds(base_row + (_c + 2) * chunk_rows, chunk_rows), :],
                    bufs[_b], sems[_b]).start()
        return 0

    lax.fori_loop(0, nchunks // 2, outer, 0)
    pltpu.sync_copy(osum, out_hbm.at[pl.ds(wid * rows_pw, rows_pw)])


def _tc_head_body(x_ref, xm_ref, sq_ref, df_ref, *, kb):
    i = pl.program_id(0)
    x = x_ref[...]
    r8 = x.shape[0] // 128
    sq_ref[...] = jnp.sum(x * x, axis=1).reshape(r8, 128)

    @pl.when(i < kb)
    def _diff():
        dx = x - xm_ref[...]
        df_ref[...] = jnp.sum(dx * dx, axis=1).reshape(r8, 128)


def _tc_final_body(sqh_ref, sqt_ref, df_ref, out_ref, *, n, k, d, k_rows):
    sqh = sqh_ref[...]
    sqt = sqt_ref[...]
    sq_all = jnp.concatenate([sqh, sqt], axis=0)
    norms = jnp.sqrt(sq_all)
    total_norm = jnp.sum(norms)
    fro2 = jnp.sum(sq_all)
    bits = lax.bitcast_convert_type(norms, jnp.int32)

    def bisect(_, lohi):
        lo, hi = lohi
        mid = lo + (hi - lo) // 2
        cnt = jnp.sum((bits >= mid).astype(jnp.int32))
        ge = cnt >= k
        return (jnp.where(ge, mid, lo), jnp.where(ge, hi, mid))

    lo, _ = lax.fori_loop(0, 31, bisect, (jnp.int32(0), jnp.int32(0x7F800000)))
    tval = jnp.max(jnp.where(bits <= lo, norms, 0.0))
    gt = bits > lo
    cnt_gt = jnp.sum(gt.astype(jnp.int32))
    sum_gt = jnp.sum(jnp.where(gt, norms, 0.0))
    top_sum = sum_gt + (k - cnt_gt).astype(jnp.float32) * tval

    gamma = top_sum / total_norm
    thr = (1.0 - gamma) * (1.0 - gamma) * fro2
    inv_d = 1.0 / d
    pen1 = jnp.sum(jnp.maximum(df_ref[...] * inv_d - thr, 0.0))
    rows_idx = lax.broadcasted_iota(jnp.int32, sqh.shape, 0)
    pen2 = jnp.sum(jnp.where(rows_idx >= k_rows,
                             jnp.maximum(sqh * inv_d - thr, 0.0), 0.0))
    pen3 = jnp.sum(jnp.maximum(sqt * inv_d - thr, 0.0))
    out_ref[0, 0] = (pen1 + pen2 + pen3) / n


def kernel(X, X_merged):
    B, N, D = X.shape
    K = X_merged.shape[1]
    top_k = min(K, N // 2)
    XTRA = 2048
    HEAD = K + XTRA
    TAIL = N - HEAD
    ROWS_PW = TAIL // _NW
    CHUNK = 32

    X2 = X.reshape(N, D)
    Xm2 = X_merged.reshape(K, D)

    mesh = plsc.VectorSubcoreMesh(core_axis_name="c", subcore_axis_name="s")
    sc_tail = functools.partial(
        pl.kernel,
        mesh=mesh,
        compiler_params=pltpu.CompilerParams(
            needs_layout_passes=False),
        out_type=jax.ShapeDtypeStruct((TAIL,), jnp.float32),
        scratch_types=[
            pltpu.VMEM((CHUNK, D), jnp.float32),
            pltpu.VMEM((CHUNK, D), jnp.float32),
            pltpu.VMEM((ROWS_PW,), jnp.float32),
            pltpu.SemaphoreType.DMA,
            pltpu.SemaphoreType.DMA,
        ],
    )(functools.partial(_sc_tail_body, head_rows=HEAD, rows_pw=ROWS_PW,
                        chunk_rows=CHUNK, d=D))
    sq_tail = sc_tail(X2)

    HB = 1024
    KB = K // HB
    sq_head, df = pl.pallas_call(
        functools.partial(_tc_head_body, kb=KB),
        grid=(HEAD // HB,),
        in_specs=[
            pl.BlockSpec((HB, D), lambda i: (i, 0)),
            pl.BlockSpec((HB, D), lambda i: (jnp.minimum(i, KB - 1), 0)),
        ],
        out_specs=[
            pl.BlockSpec((HB // 128, 128), lambda i: (i, 0)),
            pl.BlockSpec((HB // 128, 128), lambda i: (jnp.minimum(i, KB - 1), 0)),
        ],
        out_shape=[
            jax.ShapeDtypeStruct((HEAD // 128, 128), jnp.float32),
            jax.ShapeDtypeStruct((K // 128, 128), jnp.float32),
        ],
    )(X2, Xm2)

    final_body = functools.partial(_tc_final_body, n=N, k=top_k, d=float(D),
                                   k_rows=K // 128)
    out = pl.pallas_call(
        final_body,
        out_specs=pl.BlockSpec(memory_space=pltpu.SMEM),
        out_shape=jax.ShapeDtypeStruct((1, 1), jnp.float32),
    )(sq_head, sq_tail.reshape(TAIL // 128, 128), df)
    return out.reshape(())


# speedup vs baseline: 2.3566x; 1.0200x over previous
import functools

import jax
import jax.numpy as jnp
from jax import lax
from jax.experimental import pallas as pl
from jax.experimental.pallas import tpu as pltpu
from jax.experimental.pallas import tpu_sc as plsc

_NC = 1
_NS = 16
_NW = _NC * _NS
_L = 16


def _sc_tail_body(x_hbm, out_hbm, buf0, buf1, osum, sem0, sem1, *,
                  head_rows, rows_pw, chunk_rows, d):
    wid = lax.axis_index("s") * _NC + lax.axis_index("c")
    base_row = head_rows + wid * rows_pw
    nchunks = rows_pw // chunk_rows
    bufs = (buf0, buf1)
    sems = (sem0, sem1)

    for b in range(2):
        pltpu.make_async_copy(
            x_hbm.at[pl.ds(base_row + b * chunk_rows, chunk_rows), :],
            bufs[b], sems[b]).start()

    lane_ids = lax.iota(jnp.int32, _L)

    def outer(it, _):
        for b in range(2):
            c = 2 * it + b
            row0 = c * chunk_rows
            pltpu.make_async_copy(
                x_hbm.at[pl.ds(base_row + row0, chunk_rows), :],
                bufs[b], sems[b]).wait()

            def gbody(g, _, _b=b, _row0=row0):
                rowsums = jnp.zeros((_L,), jnp.float32)
                for r in range(_L):
                    rr = g * _L + r
                    acc = jnp.zeros((_L,), jnp.float32)

                    def jbody(jq, acc, _rr=rr):
                        off = jq * (_L * 32)
                        for u in range(32):
                            v = bufs[_b][_rr, pl.ds(off + u * _L, _L)]
                            acc = acc + v * v
                        return acc

                    acc = lax.fori_loop(0, d // (_L * 32), jbody, acc)
                    rowsums = jnp.where(lane_ids == r, jnp.sum(acc), rowsums)
                osum[pl.ds(_row0 + g * _L, _L)] = rowsums
                return 0

            lax.fori_loop(0, chunk_rows // _L, gbody, 0)

            @pl.when(c + 2 < nchunks)
            def _prefetch(_b=b, _c=c):
                pltpu.make_async_copy(
                    x_hbm.at[pl.ds(base_row + (_c + 2) * chunk_rows, chunk_rows), :],
                    bufs[_b], sems[_b]).start()
        return 0

    lax.fori_loop(0, nchunks // 2, outer, 0)
    pltpu.sync_copy(osum, out_hbm.at[pl.ds(wid * rows_pw, rows_pw)])


def _tc_head_body(x_ref, xm_ref, sq_ref, df_ref, *, kb):
    i = pl.program_id(0)
    x = x_ref[...]
    r8 = x.shape[0] // 128
    sq_ref[...] = jnp.sum(x * x, axis=1).reshape(r8, 128)

    @pl.when(i < kb)
    def _diff():
        dx = x - xm_ref[...]
        df_ref[...] = jnp.sum(dx * dx, axis=1).reshape(r8, 128)


def _tc_final_body(sqh_ref, sqt_ref, df_ref, out_ref, *, n, k, d, k_rows):
    sqh = sqh_ref[...]
    sqt = sqt_ref[...]
    sq_all = jnp.concatenate([sqh, sqt], axis=0)
    norms = jnp.sqrt(sq_all)
    total_norm = jnp.sum(norms)
    fro2 = jnp.sum(sq_all)
    bits = lax.bitcast_convert_type(norms, jnp.int32)

    def bisect(_, lohi):
        lo, hi = lohi
        mid = lo + (hi - lo) // 2
        cnt = jnp.sum((bits >= mid).astype(jnp.int32))
        ge = cnt >= k
        return (jnp.where(ge, mid, lo), jnp.where(ge, hi, mid))

    lo, _ = lax.fori_loop(0, 31, bisect, (jnp.int32(0), jnp.int32(0x7F800000)))
    tval = jnp.max(jnp.where(bits <= lo, norms, 0.0))
    gt = bits > lo
    cnt_gt = jnp.sum(gt.astype(jnp.int32))
    sum_gt = jnp.sum(jnp.where(gt, norms, 0.0))
    top_sum = sum_gt + (k - cnt_gt).astype(jnp.float32) * tval

    gamma = top_sum / total_norm
    thr = (1.0 - gamma) * (1.0 - gamma) * fro2
    inv_d = 1.0 / d
    pen1 = jnp.sum(jnp.maximum(df_ref[...] * inv_d - thr, 0.0))
    rows_idx = lax.broadcasted_iota(jnp.int32, sqh.shape, 0)
    pen2 = jnp.sum(jnp.where(rows_idx >= k_rows,
                             jnp.maximum(sqh * inv_d - thr, 0.0), 0.0))
    pen3 = jnp.sum(jnp.maximum(sqt * inv_d - thr, 0.0))
    out_ref[0, 0] = (pen1 + pen2 + pen3) / n


def kernel(X, X_merged):
    B, N, D = X.shape
    K = X_merged.shape[1]
    top_k = min(K, N // 2)
    XTRA = 4096
    HEAD = K + XTRA
    TAIL = N - HEAD
    ROWS_PW = TAIL // _NW
    CHUNK = 32

    X2 = X.reshape(N, D)
    Xm2 = X_merged.reshape(K, D)

    mesh = plsc.VectorSubcoreMesh(core_axis_name="c", subcore_axis_name="s",
                                  num_cores=_NC)
    sc_tail = functools.partial(
        pl.kernel,
        mesh=mesh,
        compiler_params=pltpu.CompilerParams(
            needs_layout_passes=False),
        out_type=jax.ShapeDtypeStruct((TAIL,), jnp.float32),
        scratch_types=[
            pltpu.VMEM((CHUNK, D), jnp.float32),
            pltpu.VMEM((CHUNK, D), jnp.float32),
            pltpu.VMEM((ROWS_PW,), jnp.float32),
            pltpu.SemaphoreType.DMA,
            pltpu.SemaphoreType.DMA,
        ],
    )(functools.partial(_sc_tail_body, head_rows=HEAD, rows_pw=ROWS_PW,
                        chunk_rows=CHUNK, d=D))
    sq_tail = sc_tail(X2)

    HB = 1024
    KB = K // HB
    sq_head, df = pl.pallas_call(
        functools.partial(_tc_head_body, kb=KB),
        grid=(HEAD // HB,),
        in_specs=[
            pl.BlockSpec((HB, D), lambda i: (i, 0)),
            pl.BlockSpec((HB, D), lambda i: (jnp.minimum(i, KB - 1), 0)),
        ],
        out_specs=[
            pl.BlockSpec((HB // 128, 128), lambda i: (i, 0)),
            pl.BlockSpec((HB // 128, 128), lambda i: (jnp.minimum(i, KB - 1), 0)),
        ],
        out_shape=[
            jax.ShapeDtypeStruct((HEAD // 128, 128), jnp.float32),
            jax.ShapeDtypeStruct((K // 128, 128), jnp.float32),
        ],
    )(X2, Xm2)

    final_body = functools.partial(_tc_final_body, n=N, k=top_k, d=float(D),
                                   k_rows=K // 128)
    out = pl.pallas_call(
        final_body,
        out_specs=pl.BlockSpec(memory_space=pltpu.SMEM),
        out_shape=jax.ShapeDtypeStruct((1, 1), jnp.float32),
    )(sq_head, sq_tail.reshape(TAIL // 128, 128), df)
    return out.reshape(())


# speedup vs baseline: 5.3734x; 2.2801x over previous
import functools

import jax
import jax.numpy as jnp
from jax import lax
from jax.experimental import pallas as pl
from jax.experimental.pallas import tpu as pltpu


def _body(x_ref, xm_ref, out_ref, sq_ref, df_ref, *, grid_n, rows, n, k, d, kb):
    i = pl.program_id(0)
    x = x_ref[...]
    sq = jnp.sum(x * x, axis=1)
    r8 = rows // 128
    sq_ref[pl.ds(i * r8, r8), :] = sq.reshape(r8, 128)

    @pl.when(i < kb)
    def _store_diff():
        dxy = x - xm_ref[...]
        df = jnp.sum(dxy * dxy, axis=1)
        df_ref[pl.ds(i * r8, r8), :] = df.reshape(r8, 128)

    @pl.when(i == grid_n - 1)
    def _finalize():
        sq_all = sq_ref[...]
        norms = jnp.sqrt(sq_all)
        total_norm = jnp.sum(norms)
        fro2 = jnp.sum(sq_all)
        bits = lax.bitcast_convert_type(norms, jnp.int32)

        def bisect(_, lohi):
            lo, hi = lohi
            mid = lo + (hi - lo) // 2
            cnt = jnp.sum((bits >= mid).astype(jnp.int32))
            ge = cnt >= k
            return (jnp.where(ge, mid, lo), jnp.where(ge, hi, mid))

        lo, _ = lax.fori_loop(0, 31, bisect, (jnp.int32(0), jnp.int32(0x7F800000)))
        tval = jnp.max(jnp.where(bits <= lo, norms, 0.0))
        gt = bits > lo
        cnt_gt = jnp.sum(gt.astype(jnp.int32))
        sum_gt = jnp.sum(jnp.where(gt, norms, 0.0))
        top_sum = sum_gt + (k - cnt_gt).astype(jnp.float32) * tval

        gamma = top_sum / total_norm
        thr = (1.0 - gamma) * (1.0 - gamma) * fro2
        inv_d = 1.0 / d
        pen1 = jnp.sum(jnp.maximum(df_ref[...] * inv_d - thr, 0.0))
        rows_idx = lax.broadcasted_iota(jnp.int32, sq_all.shape, 0)
        tail = jnp.where(rows_idx >= (kb * rows) // 128,
                         jnp.maximum(sq_all * inv_d - thr, 0.0), 0.0)
        pen2 = jnp.sum(tail)
        out_ref[0, 0] = (pen1 + pen2) / n


def kernel(X, X_merged):
    B, N, D = X.shape
    K = X_merged.shape[1]
    top_k = min(K, N // 2)
    ROWS = 1024
    GRID = N // ROWS
    KB = K // ROWS

    X2 = X.reshape(N, D)
    Xm2 = X_merged.reshape(K, D)

    body = functools.partial(
        _body, grid_n=GRID, rows=ROWS, n=N, k=top_k, d=float(D), kb=KB)
    out = pl.pallas_call(
        body,
        grid=(GRID,),
        in_specs=[
            pl.BlockSpec((ROWS, D), lambda i: (i, 0)),
            pl.BlockSpec((ROWS, D), lambda i: (jnp.minimum(i, KB - 1), 0)),
        ],
        out_specs=pl.BlockSpec(memory_space=pltpu.SMEM),
        out_shape=jax.ShapeDtypeStruct((1, 1), jnp.float32),
        scratch_shapes=[
            pltpu.VMEM((N // 128, 128), jnp.float32),
            pltpu.VMEM((K // 128, 128), jnp.float32),
        ],
    )(X2, Xm2)
    return out.reshape(())
